# baseline (speedup 1.0000x reference)
import functools

import jax
import jax.numpy as jnp
from jax import lax
from jax.experimental import pallas as pl
from jax.experimental.pallas import tpu as pltpu
from jax.experimental.pallas import tpu_sc as plsc

VOCAB = 1000001
B = 16384
L = 50
D = 300
C = 2 * L
CP = 128


_BM = 8192


def _mm_body(t_ref, w_ref, p_ref):
    p_ref[...] = jnp.dot(
        t_ref[...].astype(jnp.bfloat16),
        w_ref[...],
        preferred_element_type=jnp.float32,
    )


def _project_table(table, wt_bf16):
    grid = (pl.cdiv(VOCAB, _BM),)
    return pl.pallas_call(
        _mm_body,
        grid=grid,
        in_specs=[
            pl.BlockSpec((_BM, D), lambda i: (i, 0)),
            pl.BlockSpec((D, CP), lambda i: (0, 0)),
        ],
        out_specs=pl.BlockSpec((_BM, CP), lambda i: (i, 0)),
        out_shape=jax.ShapeDtypeStruct((VOCAB, CP), jnp.float32),
    )(table, wt_bf16)


_NC = 2
_NS = 16
_NW = _NC * _NS
_CH = B // _NW
_VB = 16


def _sc_body(idxt_hbm, pflat_hbm, b_hbm, out_hbm,
             i_a, i_b, v_a, v_b, acc, bvm,
             sem_ia, sem_ib, sem_ga, sem_gb):
    wid = lax.axis_index("s") * _NC + lax.axis_index("c")
    base = wid * _CH

    def addr(buf, c):
        for j in range(_CH // _VB):
            s = pl.ds(j * _VB, _VB)
            buf[s] = buf[s] * CP + c

    def accum(vbuf):
        for j in range(_CH // _VB):
            s = pl.ds(j * _VB, _VB)
            acc[s] = acc[s] + vbuf[s]

    def wait_idx(buf, sem):
        pltpu.make_async_copy(idxt_hbm.at[0, pl.ds(base, _CH)], buf, sem).wait()

    def wait_g(vbuf, sem):
        pltpu.make_async_copy(idxt_hbm.at[0, pl.ds(base, _CH)], vbuf, sem).wait()

    pltpu.sync_copy(b_hbm, bvm)
    bvec = bvm[...]
    for j in range(_CH // _VB):
        acc[pl.ds(j * _VB, _VB)] = bvec

    pltpu.sync_copy(idxt_hbm.at[0, pl.ds(base, _CH)], i_a)
    addr(i_a, 0)
    pltpu.async_copy(pflat_hbm.at[i_a], v_a, sem_ga)
    pltpu.async_copy(idxt_hbm.at[1, pl.ds(base, _CH)], i_b, sem_ib)

    def step(i, _):
        c0 = 2 * i
        more = i < (C // 2 - 1)
        wait_idx(i_b, sem_ib)
        addr(i_b, c0 + 1)
        wait_g(v_a, sem_ga)
        accum(v_a)
        pltpu.async_copy(pflat_hbm.at[i_b], v_b, sem_gb)

        @pl.when(more)
        def _():
            pltpu.async_copy(idxt_hbm.at[c0 + 2, pl.ds(base, _CH)], i_a, sem_ia)
            wait_idx(i_a, sem_ia)
            addr(i_a, c0 + 2)

        wait_g(v_b, sem_gb)
        accum(v_b)

        @pl.when(more)
        def _():
            pltpu.async_copy(pflat_hbm.at[i_a], v_a, sem_ga)
            pltpu.async_copy(idxt_hbm.at[c0 + 3, pl.ds(base, _CH)], i_b, sem_ib)

        return 0

    lax.fori_loop(0, C // 2, step, 0, unroll=False)

    for j in range(_CH // _VB):
        s = pl.ds(j * _VB, _VB)
        x = acc[s]
        acc[s] = 1.0 / (1.0 + jnp.exp(-x))

    pltpu.sync_copy(acc, out_hbm.at[pl.ds(base, _CH)])


@functools.cache
def _sc_gather():
    return pl.kernel(
        _sc_body,
        out_type=jax.ShapeDtypeStruct((B,), jnp.float32),
        mesh=plsc.VectorSubcoreMesh(
            core_axis_name="c", subcore_axis_name="s",
            num_cores=_NC, num_subcores=_NS,
        ),
        scratch_types=[
            pltpu.VMEM((_CH,), jnp.int32),
            pltpu.VMEM((_CH,), jnp.int32),
            pltpu.VMEM((_CH,), jnp.float32),
            pltpu.VMEM((_CH,), jnp.float32),
            pltpu.VMEM((_CH,), jnp.float32),
            pltpu.VMEM((_VB,), jnp.float32),
            pltpu.SemaphoreType.DMA,
            pltpu.SemaphoreType.DMA,
            pltpu.SemaphoreType.DMA,
            pltpu.SemaphoreType.DMA,
        ],
    )


def kernel(input1, input2, table, W, b):
    wt = W[:, 0].reshape(C, D).T.astype(jnp.bfloat16)
    wt = jnp.pad(wt, ((0, 0), (0, CP - C)))
    p = _project_table(table, wt)
    pflat = p.reshape(VOCAB * CP)
    idx_all = jnp.concatenate(
        [input1, input2, jnp.zeros((B, CP - C), jnp.int32)], axis=1
    )
    idxt = idx_all.T
    bvec = jnp.broadcast_to(b, (_VB,))
    out = _sc_gather()(idxt, pflat, bvec)
    return out.reshape(B, 1)


# speedup vs baseline: 1.0019x; 1.0019x over previous
import functools

import jax
import jax.numpy as jnp
from jax import lax
from jax.experimental import pallas as pl
from jax.experimental.pallas import tpu as pltpu
from jax.experimental.pallas import tpu_sc as plsc

VOCAB = 1000001
B = 16384
L = 50
D = 300
C = 2 * L
CP = 128


_BM = 8192
_CHK = 2048
_NSLOT = 8
_NSTEP = 122
_TAILBLK = _NSTEP * _BM // 1024


def _mm_body(t_ref, tail_ref, w_ref, p_ref, *scr):
    bufs, sems = scr[:_NSLOT], scr[_NSLOT:]
    i = pl.program_id(0)

    def fire(c, slot):
        src = t_ref.at[pl.ds(c * _CHK, _CHK), :]
        pltpu.make_async_copy(src, bufs[slot], sems[slot]).start()

    @pl.when(i == 0)
    def _():
        for s in range(_NSLOT):
            fire(s, s)

    def consume(base_slot):
        for k in range(4):
            slot = base_slot + k
            c = 4 * i + k
            pltpu.make_async_copy(
                t_ref.at[pl.ds(0, _CHK), :], bufs[slot], sems[slot]
            ).wait()
            p_ref[pl.ds(k * _CHK, _CHK), :] = jnp.dot(
                bufs[slot][...].astype(jnp.bfloat16),
                w_ref[...],
                preferred_element_type=jnp.float32,
            )

            @pl.when(c + _NSLOT < 4 * _NSTEP)
            def _():
                fire(c + _NSLOT, slot)

    @pl.when(jnp.logical_and(i < _NSTEP, lax.rem(i, 2) == 0))
    def _():
        consume(0)

    @pl.when(jnp.logical_and(i < _NSTEP, lax.rem(i, 2) == 1))
    def _():
        consume(4)

    @pl.when(i == _NSTEP)
    def _():
        p_ref[pl.ds(0, 1024), :] = jnp.dot(
            tail_ref[...].astype(jnp.bfloat16),
            w_ref[...],
            preferred_element_type=jnp.float32,
        )


def _project_table(table, wt_bf16):
    grid = (_NSTEP + 1,)
    in_specs = [
        pl.BlockSpec(memory_space=pl.ANY),
        pl.BlockSpec((1024, D), lambda i: (_TAILBLK, 0)),
        pl.BlockSpec((D, CP), lambda i: (0, 0)),
    ]
    scratch = [pltpu.VMEM((_CHK, D), jnp.float32) for _ in range(_NSLOT)]
    scratch += [pltpu.SemaphoreType.DMA for _ in range(_NSLOT)]
    return pl.pallas_call(
        _mm_body,
        grid=grid,
        in_specs=in_specs,
        out_specs=pl.BlockSpec((_BM, CP), lambda i: (i, 0)),
        out_shape=jax.ShapeDtypeStruct((VOCAB, CP), jnp.float32),
        scratch_shapes=scratch,
    )(table, table, wt_bf16)


_NC = 2
_NS = 16
_NW = _NC * _NS
_CH = B // _NW
_VB = 16


def _sc_body(idxt_hbm, pflat_hbm, b_hbm, out_hbm,
             i_a, i_b, v_a, v_b, acc, bvm,
             sem_ia, sem_ib, sem_ga, sem_gb):
    wid = lax.axis_index("s") * _NC + lax.axis_index("c")
    base = wid * _CH

    def addr(buf, c):
        for j in range(_CH // _VB):
            s = pl.ds(j * _VB, _VB)
            buf[s] = buf[s] * CP + c

    def accum(vbuf):
        for j in range(_CH // _VB):
            s = pl.ds(j * _VB, _VB)
            acc[s] = acc[s] + vbuf[s]

    def wait_idx(buf, sem):
        pltpu.make_async_copy(idxt_hbm.at[0, pl.ds(base, _CH)], buf, sem).wait()

    def wait_g(vbuf, sem):
        pltpu.make_async_copy(idxt_hbm.at[0, pl.ds(base, _CH)], vbuf, sem).wait()

    pltpu.sync_copy(b_hbm, bvm)
    bvec = bvm[...]
    for j in range(_CH // _VB):
        acc[pl.ds(j * _VB, _VB)] = bvec

    pltpu.sync_copy(idxt_hbm.at[0, pl.ds(base, _CH)], i_a)
    addr(i_a, 0)
    pltpu.async_copy(pflat_hbm.at[i_a], v_a, sem_ga)
    pltpu.async_copy(idxt_hbm.at[1, pl.ds(base, _CH)], i_b, sem_ib)

    def step(i, _):
        c0 = 2 * i
        more = i < (C // 2 - 1)
        wait_idx(i_b, sem_ib)
        addr(i_b, c0 + 1)
        wait_g(v_a, sem_ga)
        accum(v_a)
        pltpu.async_copy(pflat_hbm.at[i_b], v_b, sem_gb)

        @pl.when(more)
        def _():
            pltpu.async_copy(idxt_hbm.at[c0 + 2, pl.ds(base, _CH)], i_a, sem_ia)
            wait_idx(i_a, sem_ia)
            addr(i_a, c0 + 2)

        wait_g(v_b, sem_gb)
        accum(v_b)

        @pl.when(more)
        def _():
            pltpu.async_copy(pflat_hbm.at[i_a], v_a, sem_ga)
            pltpu.async_copy(idxt_hbm.at[c0 + 3, pl.ds(base, _CH)], i_b, sem_ib)

        return 0

    lax.fori_loop(0, C // 2, step, 0, unroll=False)

    for j in range(_CH // _VB):
        s = pl.ds(j * _VB, _VB)
        x = acc[s]
        acc[s] = 1.0 / (1.0 + jnp.exp(-x))

    pltpu.sync_copy(acc, out_hbm.at[pl.ds(base, _CH)])


@functools.cache
def _sc_gather():
    return pl.kernel(
        _sc_body,
        out_type=jax.ShapeDtypeStruct((B,), jnp.float32),
        mesh=plsc.VectorSubcoreMesh(
            core_axis_name="c", subcore_axis_name="s",
            num_cores=_NC, num_subcores=_NS,
        ),
        scratch_types=[
            pltpu.VMEM((_CH,), jnp.int32),
            pltpu.VMEM((_CH,), jnp.int32),
            pltpu.VMEM((_CH,), jnp.float32),
            pltpu.VMEM((_CH,), jnp.float32),
            pltpu.VMEM((_CH,), jnp.float32),
            pltpu.VMEM((_VB,), jnp.float32),
            pltpu.SemaphoreType.DMA,
            pltpu.SemaphoreType.DMA,
            pltpu.SemaphoreType.DMA,
            pltpu.SemaphoreType.DMA,
        ],
    )


def kernel(input1, input2, table, W, b):
    wt = W[:, 0].reshape(C, D).T.astype(jnp.bfloat16)
    wt = jnp.pad(wt, ((0, 0), (0, CP - C)))
    p = _project_table(table, wt)
    pflat = p.reshape(VOCAB * CP)
    idx_all = jnp.concatenate(
        [input1, input2, jnp.zeros((B, CP - C), jnp.int32)], axis=1
    )
    idxt = idx_all.T
    bvec = jnp.broadcast_to(b, (_VB,))
    out = _sc_gather()(idxt, pflat, bvec)
    return out.reshape(B, 1)
